# baseline (speedup 1.0000x reference)
import functools

import jax
import jax.numpy as jnp
from jax import lax
from jax.experimental import pallas as pl
from jax.experimental.pallas import tpu as pltpu
from jax.experimental.pallas import tpu_sc as plsc

_N = 1024
_K = 64
_DIM = 128
_L = 2
_NREL = 32

_NC, _NS = 2, 16
_NW = _NC * _NS
_R = 2 * _N * _K * (1 + 2 * _L)
_PW = _R // _NW
_CH = 128
_NCHUNK = _PW // (2 * _CH)

_BP = 64
_RB = _BP * _K
_RB2 = _RB // 2


_HD = _DIM // 2


def _sc_gather(table, idx):
    mesh = plsc.VectorSubcoreMesh(
        core_axis_name="c", subcore_axis_name="s",
        num_cores=_NC, num_subcores=_NS)

    @functools.partial(
        pl.kernel,
        out_type=jax.ShapeDtypeStruct((_R // 2, _DIM), jnp.int32),
        mesh=mesh,
        compiler_params=pltpu.CompilerParams(use_tc_tiling_on_sc=False),
        scratch_types=[
            pltpu.VMEM((2, _CH), jnp.int32),
            pltpu.VMEM((2, _CH), jnp.int32),
            pltpu.VMEM((_CH, _HD), jnp.int32),
            pltpu.VMEM((_CH, _HD), jnp.int32),
            pltpu.VMEM((_CH, _HD), jnp.int32),
            pltpu.VMEM((_CH, _HD), jnp.int32),
            pltpu.SemaphoreType.DMA,
            pltpu.SemaphoreType.DMA,
            pltpu.SemaphoreType.DMA,
            pltpu.SemaphoreType.DMA,
            pltpu.SemaphoreType.DMA,
            pltpu.SemaphoreType.DMA,
        ],
    )
    def k(table_hbm, idx_hbm, out_hbm, idxv0, idxv1, re0, ro0, re1, ro1,
          isem0, isem1, gsem0, gsem1, ssem0, ssem1):
        wid = lax.axis_index("s") * _NC + lax.axis_index("c")
        cbase = wid * _NCHUNK
        idxv = (idxv0, idxv1)
        rowe = (re0, re1)
        rowo = (ro0, ro1)
        isem = (isem0, isem1)
        gsem = (gsem0, gsem1)
        ssem = (ssem0, ssem1)

        def istart(i, b):
            pltpu.async_copy(idx_hbm.at[cbase + i], idxv[b], isem[b])

        def iwait(i, b):
            pltpu.make_async_copy(idx_hbm.at[cbase + i], idxv[b],
                                  isem[b]).wait()

        def gstart(b):
            pltpu.async_copy(table_hbm.at[idxv[b].at[0]], rowe[b], gsem[b])
            pltpu.async_copy(table_hbm.at[idxv[b].at[1]], rowo[b], gsem[b])

        def gwait(b):
            pltpu.make_async_copy(table_hbm.at[idxv[b].at[0]], rowe[b],
                                  gsem[b]).wait()
            pltpu.make_async_copy(table_hbm.at[idxv[b].at[1]], rowo[b],
                                  gsem[b]).wait()

        def sstart(i, b):
            orow = (cbase + i) * _CH
            pltpu.async_copy(rowe[b], out_hbm.at[pl.ds(orow, _CH), 0:_HD],
                             ssem[b])
            pltpu.async_copy(rowo[b], out_hbm.at[pl.ds(orow, _CH), _HD:_DIM],
                             ssem[b])

        def swait(i, b):
            orow = (cbase + i) * _CH
            pltpu.make_async_copy(rowe[b],
                                  out_hbm.at[pl.ds(orow, _CH), 0:_HD],
                                  ssem[b]).wait()
            pltpu.make_async_copy(rowo[b],
                                  out_hbm.at[pl.ds(orow, _CH), _HD:_DIM],
                                  ssem[b]).wait()

        istart(0, 0)
        istart(1, 1)
        iwait(0, 0)
        gstart(0)
        iwait(1, 1)
        gstart(1)
        gwait(0)
        sstart(0, 0)
        istart(2, 0)

        def body(j, carry):
            i0 = 2 * j
            i1 = i0 + 1
            swait(i0 - 2, 0)
            iwait(i0, 0)
            gstart(0)
            gwait(1)
            sstart(i0 - 1, 1)
            istart(i1, 1)
            swait(i1 - 2, 1)
            iwait(i1, 1)
            gstart(1)
            gwait(0)
            sstart(i0, 0)
            istart(i1 + 1, 0)
            return carry

        lax.fori_loop(1, _NCHUNK // 2 - 1, body, 0)

        i0 = _NCHUNK - 2
        i1 = _NCHUNK - 1
        swait(i0 - 2, 0)
        iwait(i0, 0)
        gstart(0)
        gwait(1)
        sstart(i0 - 1, 1)
        istart(i1, 1)
        swait(i1 - 2, 1)
        iwait(i1, 1)
        gstart(1)
        gwait(0)
        sstart(i0, 0)
        gwait(1)
        sstart(i1, 1)
        swait(i0, 0)
        swait(i1, 1)

    return k(table, idx)


def _unpack2(w):
    f32 = jnp.float32
    lo = jax.lax.bitcast_convert_type(jnp.left_shift(w, 16), f32)
    hi = jax.lax.bitcast_convert_type(
        jnp.bitwise_and(w, jnp.int32(-65536)), f32)
    return lo, hi


def _attn_block(h_ref, t_ref, rele_ref, relo_ref, re_ref, w1_ref, b1_ref,
                w2_ref, b2_ref, w3_ref, b3_ref, out_ref):
    f32 = jnp.float32
    lo, hi = _unpack2(h_ref[0])
    w1b = w1_ref[_DIM:2 * _DIM, :]
    rt = jnp.dot(re_ref[...], w1b, preferred_element_type=f32)
    a0 = w1_ref[0:_HD, :]
    a1 = w1_ref[_HD:_DIM, :]
    cls = jax.lax.broadcasted_iota(jnp.int32, (_NREL, _RB2), 0)

    def stream(hl, hh, relv):
        oht = jnp.where(relv == cls, 1.0, 0.0)
        rb = jax.lax.dot_general(oht, rt, (((0,), (0,)), ((), ())),
                                 preferred_element_type=f32)
        y1 = (jnp.dot(hl, a0, preferred_element_type=f32)
              + jnp.dot(hh, a1, preferred_element_type=f32) + rb + b1_ref[...])
        y1 = jnp.maximum(y1, 0.0)
        y2 = jnp.maximum(jnp.dot(y1, w2_ref[...], preferred_element_type=f32)
                         + b2_ref[...], 0.0)
        s = jax.nn.sigmoid(jnp.dot(y2, w3_ref[...],
                                   preferred_element_type=f32) + b3_ref[...])
        return jnp.exp(s[:, 0:1])

    e_even = stream(lo[:, 0:_HD], hi[:, 0:_HD], rele_ref[0])
    e_odd = stream(lo[:, _HD:_DIM], hi[:, _HD:_DIM], relo_ref[0])

    tlo, thi = _unpack2(t_ref[0])
    den = jnp.sum((e_even + e_odd).reshape(_BP, _K // 2, 1), axis=1)
    nlo = jnp.sum((e_even * tlo[:, 0:_HD]
                   + e_odd * tlo[:, _HD:_DIM]).reshape(_BP, _K // 2, _HD),
                  axis=1)
    nhi = jnp.sum((e_even * thi[:, 0:_HD]
                   + e_odd * thi[:, _HD:_DIM]).reshape(_BP, _K // 2, _HD),
                  axis=1)
    out_ref[0, :, 0:_HD] = nlo / den
    out_ref[0, :, _HD:_DIM] = nhi / den


def _mean_block(x_ref, out_ref):
    xlo, xhi = _unpack2(x_ref[0])
    out_ref[0, :, 0:_HD] = jnp.sum(
        (xlo[:, 0:_HD] + xlo[:, _HD:_DIM]).reshape(_BP, _K // 2, _HD),
        axis=1) * (1.0 / _K)
    out_ref[0, :, _HD:_DIM] = jnp.sum(
        (xhi[:, 0:_HD] + xhi[:, _HD:_DIM]).reshape(_BP, _K // 2, _HD),
        axis=1) * (1.0 / _K)


def _agg_block(emu_ref, pu0_ref, pu1_ref, emi_ref, pi0_ref, pi1_ref,
               wagg_ref, bagg_ref, out_ref):
    f32 = jnp.float32
    wg0 = wagg_ref[0:_DIM, :]
    wg1 = wagg_ref[_DIM:2 * _DIM, :]
    wg2 = wagg_ref[2 * _DIM:3 * _DIM, :]
    b = bagg_ref[...]
    ue = jax.nn.sigmoid(
        jnp.dot(emu_ref[...], wg0, preferred_element_type=f32)
        + jnp.dot(pu0_ref[...], wg1, preferred_element_type=f32)
        + jnp.dot(pu1_ref[...], wg2, preferred_element_type=f32) + b)
    ie = jax.nn.sigmoid(
        jnp.dot(emi_ref[...], wg0, preferred_element_type=f32)
        + jnp.dot(pi0_ref[...], wg1, preferred_element_type=f32)
        + jnp.dot(pi1_ref[...], wg2, preferred_element_type=f32) + b)
    out_ref[...] = jax.nn.sigmoid(jnp.sum(ue * ie, axis=1, keepdims=True))


def kernel(u_entity, u_heads, u_relations, u_tails,
           i_entity, i_heads, i_relations, i_tails,
           entity_emb, rel_emb, W1, b1, W2, b2, W3, b3, Wagg, bagg):
    f32 = jnp.float32
    i32 = jnp.int32
    nent_rows = 2 * _N * _K
    nhead_rows = 2 * _L * _N * _K

    idx = jnp.concatenate([
        u_entity.reshape(-1), i_entity.reshape(-1),
        u_heads.reshape(-1), i_heads.reshape(-1),
        u_tails.reshape(-1), i_tails.reshape(-1)]).astype(i32)
    v = idx.reshape(-1, _CH, 2)
    idx_r = jnp.stack([v[..., 0], v[..., 1]], axis=1)

    ebf = entity_emb.astype(jnp.bfloat16)
    lo16 = jax.lax.bitcast_convert_type(ebf[:, :_HD], jnp.uint16).astype(i32)
    hi16 = jax.lax.bitcast_convert_type(ebf[:, _HD:], jnp.uint16).astype(i32)
    tbl = jnp.bitwise_or(lo16, jnp.left_shift(hi16, 16))

    g = _sc_gather(tbl, idx_r)

    ne2 = nent_rows // 2
    nh2 = nhead_rows // 2
    ent_rows = g[:ne2].reshape(-1, _RB2, _DIM)
    head_rows = g[ne2:ne2 + nh2].reshape(-1, _RB2, _DIM)
    tail_rows = g[ne2 + nh2:].reshape(-1, _RB2, _DIM)

    rel = jnp.concatenate([u_relations, i_relations], axis=0).astype(i32)
    rel_even = rel[..., 0::2].reshape(-1, 1, _RB2)
    rel_odd = rel[..., 1::2].reshape(-1, 1, _RB2)

    w3p = jnp.pad(W3, ((0, 0), (0, _DIM - 1)))
    b3p = jnp.pad(b3.reshape(1, 1), ((0, 0), (0, _DIM - 1)))
    nblk = head_rows.shape[0]

    pooled = pl.pallas_call(
        _attn_block,
        grid=(nblk,),
        in_specs=[
            pl.BlockSpec((1, _RB2, _DIM), lambda i: (i, 0, 0)),
            pl.BlockSpec((1, _RB2, _DIM), lambda i: (i, 0, 0)),
            pl.BlockSpec((1, 1, _RB2), lambda i: (i, 0, 0)),
            pl.BlockSpec((1, 1, _RB2), lambda i: (i, 0, 0)),
            pl.BlockSpec((_NREL, _DIM), lambda i: (0, 0)),
            pl.BlockSpec((2 * _DIM, _DIM), lambda i: (0, 0)),
            pl.BlockSpec((1, _DIM), lambda i: (0, 0)),
            pl.BlockSpec((_DIM, _DIM), lambda i: (0, 0)),
            pl.BlockSpec((1, _DIM), lambda i: (0, 0)),
            pl.BlockSpec((_DIM, _DIM), lambda i: (0, 0)),
            pl.BlockSpec((1, _DIM), lambda i: (0, 0)),
        ],
        out_specs=pl.BlockSpec((1, _BP, _DIM), lambda i: (i, 0, 0)),
        out_shape=jax.ShapeDtypeStruct((nblk, _BP, _DIM), f32),
    )(head_rows, tail_rows, rel_even, rel_odd, rel_emb, W1,
      b1.reshape(1, _DIM), W2, b2.reshape(1, _DIM), w3p, b3p)

    eblk = ent_rows.shape[0]
    means = pl.pallas_call(
        _mean_block,
        grid=(eblk,),
        in_specs=[pl.BlockSpec((1, _RB2, _DIM), lambda i: (i, 0, 0))],
        out_specs=pl.BlockSpec((1, _BP, _DIM), lambda i: (i, 0, 0)),
        out_shape=jax.ShapeDtypeStruct((eblk, _BP, _DIM), f32),
    )(ent_rows)

    means = means.reshape(2, _N, _DIM)
    pooled = pooled.reshape(2 * _L, _N, _DIM)

    out = pl.pallas_call(
        _agg_block,
        in_specs=[pl.BlockSpec((_N, _DIM), lambda: (0, 0))] * 6
        + [pl.BlockSpec(((_L + 1) * _DIM, _DIM), lambda: (0, 0)),
           pl.BlockSpec((1, _DIM), lambda: (0, 0))],
        out_specs=pl.BlockSpec((_N, 1), lambda: (0, 0)),
        out_shape=jax.ShapeDtypeStruct((_N, 1), f32),
    )(means[0], pooled[0], pooled[1], means[1], pooled[2], pooled[3],
      Wagg, bagg.reshape(1, _DIM))

    return out.reshape(_N)


# speedup vs baseline: 1.7501x; 1.7501x over previous
import functools

import jax
import jax.numpy as jnp
from jax import lax
from jax.experimental import pallas as pl
from jax.experimental.pallas import tpu as pltpu
from jax.experimental.pallas import tpu_sc as plsc

_N = 1024
_K = 64
_DIM = 128
_L = 2
_NREL = 32

_NC, _NS = 2, 16
_NW = _NC * _NS
_R = 2 * _N * _K * (1 + 2 * _L)
_PW = _R // _NW
_CH = 128
_NCHUNK = _PW // _CH

_BP = 64
_RB = _BP * _K


def _sc_gather(table, idx):
    mesh = plsc.VectorSubcoreMesh(
        core_axis_name="c", subcore_axis_name="s",
        num_cores=_NC, num_subcores=_NS)

    @functools.partial(
        pl.kernel,
        out_type=jax.ShapeDtypeStruct((_R, _DIM), jnp.float32),
        mesh=mesh,
        scratch_types=[
            pltpu.VMEM((4, _CH), jnp.int32),
            pltpu.VMEM((_CH, _DIM), jnp.float32),
            pltpu.VMEM((_CH, _DIM), jnp.float32),
            pltpu.VMEM((_CH, _DIM), jnp.float32),
            pltpu.VMEM((_CH, _DIM), jnp.float32),
            pltpu.SemaphoreType.DMA,
            pltpu.SemaphoreType.DMA,
            pltpu.SemaphoreType.DMA,
            pltpu.SemaphoreType.DMA,
            pltpu.SemaphoreType.DMA,
            pltpu.SemaphoreType.DMA,
            pltpu.SemaphoreType.DMA,
            pltpu.SemaphoreType.DMA,
            pltpu.SemaphoreType.DMA,
            pltpu.SemaphoreType.DMA,
            pltpu.SemaphoreType.DMA,
            pltpu.SemaphoreType.DMA,
        ],
    )
    def k(table_hbm, idx_hbm, out_hbm, idx_v, r0, r1, r2, r3,
          is0, is1, is2, is3, gs0, gs1, gs2, gs3, ss0, ss1, ss2, ss3):
        wid = lax.axis_index("s") * _NC + lax.axis_index("c")
        base = wid * _PW
        rows = (r0, r1, r2, r3)
        isem = (is0, is1, is2, is3)
        gsem = (gs0, gs1, gs2, gs3)
        ssem = (ss0, ss1, ss2, ss3)

        def istart(i, b):
            pltpu.async_copy(idx_hbm.at[pl.ds(base + i * _CH, _CH)],
                             idx_v.at[b], isem[b])

        def iwait(i, b):
            pltpu.make_async_copy(idx_hbm.at[pl.ds(base + i * _CH, _CH)],
                                  idx_v.at[b], isem[b]).wait()

        def gstart(b):
            pltpu.async_copy(table_hbm.at[idx_v.at[b]], rows[b], gsem[b])

        def gwait(b):
            pltpu.make_async_copy(table_hbm.at[idx_v.at[b]], rows[b],
                                  gsem[b]).wait()

        def sstart(i, b):
            pltpu.async_copy(rows[b],
                             out_hbm.at[pl.ds(base + i * _CH, _CH)], ssem[b])

        def swait(i, b):
            pltpu.make_async_copy(rows[b],
                                  out_hbm.at[pl.ds(base + i * _CH, _CH)],
                                  ssem[b]).wait()

        istart(0, 0)
        istart(1, 1)
        iwait(0, 0)
        gstart(0)
        istart(2, 2)
        iwait(1, 1)
        gstart(1)
        istart(3, 3)
        iwait(2, 2)
        gstart(2)
        gwait(0)
        sstart(0, 0)
        istart(4, 0)
        iwait(3, 3)
        gstart(3)
        gwait(1)
        sstart(1, 1)
        istart(5, 1)

        def quad(j, carry):
            i0 = 4 * j
            for b in range(4):
                i = i0 + b
                pb = (b + 2) % 4
                swait(i - 4, b)
                iwait(i, b)
                gstart(b)
                gwait(pb)
                sstart(i - 2, pb)
                istart(i + 2, pb)
            return carry

        lax.fori_loop(1, _NCHUNK // 4 - 1, quad, 0)

        for i in range(_NCHUNK - 4, _NCHUNK):
            b = i % 4
            pb = (b + 2) % 4
            swait(i - 4, b)
            iwait(i, b)
            gstart(b)
            gwait(pb)
            sstart(i - 2, pb)
            if i + 2 < _NCHUNK:
                istart(i + 2, pb)
        gwait((_NCHUNK - 2) % 4)
        sstart(_NCHUNK - 2, (_NCHUNK - 2) % 4)
        gwait((_NCHUNK - 1) % 4)
        sstart(_NCHUNK - 1, (_NCHUNK - 1) % 4)
        for i in range(_NCHUNK - 4, _NCHUNK):
            swait(i, i % 4)

    return k(table, idx)


def _attn_block(h_ref, t_ref, rel_ref, re_ref, w1_ref, b1_ref, w2_ref, b2_ref,
                w3_ref, b3_ref, out_ref):
    f32 = jnp.float32
    h = h_ref[0]
    t = t_ref[0]
    relv = rel_ref[0]
    w1a = w1_ref[0:_DIM, :]
    w1b = w1_ref[_DIM:2 * _DIM, :]
    rt = jnp.dot(re_ref[...], w1b, preferred_element_type=f32)
    cls = jax.lax.broadcasted_iota(jnp.int32, (_NREL, _RB), 0)
    oht = jnp.where(relv == cls, 1.0, 0.0)
    rb = jax.lax.dot_general(oht, rt, (((0,), (0,)), ((), ())),
                             preferred_element_type=f32)
    y1 = jnp.dot(h, w1a, preferred_element_type=f32) + rb + b1_ref[...]
    y1 = jnp.maximum(y1, 0.0)
    y2 = jnp.maximum(jnp.dot(y1, w2_ref[...], preferred_element_type=f32)
                     + b2_ref[...], 0.0)
    s = jax.nn.sigmoid(jnp.dot(y2, w3_ref[...], preferred_element_type=f32)
                       + b3_ref[...])
    e = jnp.exp(s[:, 0:1])
    num = jnp.sum((e * t).reshape(_BP, _K, _DIM), axis=1)
    den = jnp.sum(e.reshape(_BP, _K, 1), axis=1)
    out_ref[0] = num / den


def _mean_block(x_ref, out_ref):
    x = x_ref[0]
    out_ref[0] = jnp.sum(x.reshape(_BP, _K, _DIM), axis=1) * (1.0 / _K)


def _agg_block(emu_ref, pu0_ref, pu1_ref, emi_ref, pi0_ref, pi1_ref,
               wagg_ref, bagg_ref, out_ref):
    f32 = jnp.float32
    wg0 = wagg_ref[0:_DIM, :]
    wg1 = wagg_ref[_DIM:2 * _DIM, :]
    wg2 = wagg_ref[2 * _DIM:3 * _DIM, :]
    b = bagg_ref[...]
    ue = jax.nn.sigmoid(
        jnp.dot(emu_ref[...], wg0, preferred_element_type=f32)
        + jnp.dot(pu0_ref[...], wg1, preferred_element_type=f32)
        + jnp.dot(pu1_ref[...], wg2, preferred_element_type=f32) + b)
    ie = jax.nn.sigmoid(
        jnp.dot(emi_ref[...], wg0, preferred_element_type=f32)
        + jnp.dot(pi0_ref[...], wg1, preferred_element_type=f32)
        + jnp.dot(pi1_ref[...], wg2, preferred_element_type=f32) + b)
    out_ref[...] = jax.nn.sigmoid(jnp.sum(ue * ie, axis=1, keepdims=True))


def kernel(u_entity, u_heads, u_relations, u_tails,
           i_entity, i_heads, i_relations, i_tails,
           entity_emb, rel_emb, W1, b1, W2, b2, W3, b3, Wagg, bagg):
    f32 = jnp.float32
    i32 = jnp.int32
    nent_rows = 2 * _N * _K
    nhead_rows = 2 * _L * _N * _K

    idx = jnp.concatenate([
        u_entity.reshape(-1), i_entity.reshape(-1),
        u_heads.reshape(-1), i_heads.reshape(-1),
        u_tails.reshape(-1), i_tails.reshape(-1)]).astype(i32)

    g = _sc_gather(entity_emb, idx)

    ent_rows = g[:nent_rows].reshape(-1, _RB, _DIM)
    head_rows = g[nent_rows:nent_rows + nhead_rows].reshape(-1, _RB, _DIM)
    tail_rows = g[nent_rows + nhead_rows:].reshape(-1, _RB, _DIM)

    rel = jnp.concatenate([u_relations, i_relations], axis=0)
    rel = rel.reshape(-1, 1, _RB).astype(i32)

    w3p = jnp.pad(W3, ((0, 0), (0, _DIM - 1)))
    b3p = jnp.pad(b3.reshape(1, 1), ((0, 0), (0, _DIM - 1)))
    nblk = head_rows.shape[0]

    pooled = pl.pallas_call(
        _attn_block,
        grid=(nblk,),
        in_specs=[
            pl.BlockSpec((1, _RB, _DIM), lambda i: (i, 0, 0)),
            pl.BlockSpec((1, _RB, _DIM), lambda i: (i, 0, 0)),
            pl.BlockSpec((1, 1, _RB), lambda i: (i, 0, 0)),
            pl.BlockSpec((_NREL, _DIM), lambda i: (0, 0)),
            pl.BlockSpec((2 * _DIM, _DIM), lambda i: (0, 0)),
            pl.BlockSpec((1, _DIM), lambda i: (0, 0)),
            pl.BlockSpec((_DIM, _DIM), lambda i: (0, 0)),
            pl.BlockSpec((1, _DIM), lambda i: (0, 0)),
            pl.BlockSpec((_DIM, _DIM), lambda i: (0, 0)),
            pl.BlockSpec((1, _DIM), lambda i: (0, 0)),
        ],
        out_specs=pl.BlockSpec((1, _BP, _DIM), lambda i: (i, 0, 0)),
        out_shape=jax.ShapeDtypeStruct((nblk, _BP, _DIM), f32),
    )(head_rows, tail_rows, rel, rel_emb, W1, b1.reshape(1, _DIM), W2,
      b2.reshape(1, _DIM), w3p, b3p)

    eblk = ent_rows.shape[0]
    means = pl.pallas_call(
        _mean_block,
        grid=(eblk,),
        in_specs=[pl.BlockSpec((1, _RB, _DIM), lambda i: (i, 0, 0))],
        out_specs=pl.BlockSpec((1, _BP, _DIM), lambda i: (i, 0, 0)),
        out_shape=jax.ShapeDtypeStruct((eblk, _BP, _DIM), f32),
    )(ent_rows)

    means = means.reshape(2, _N, _DIM)
    pooled = pooled.reshape(2 * _L, _N, _DIM)

    out = pl.pallas_call(
        _agg_block,
        in_specs=[pl.BlockSpec((_N, _DIM), lambda: (0, 0))] * 6
        + [pl.BlockSpec(((_L + 1) * _DIM, _DIM), lambda: (0, 0)),
           pl.BlockSpec((1, _DIM), lambda: (0, 0))],
        out_specs=pl.BlockSpec((_N, 1), lambda: (0, 0)),
        out_shape=jax.ShapeDtypeStruct((_N, 1), f32),
    )(means[0], pooled[0], pooled[1], means[1], pooled[2], pooled[3],
      Wagg, bagg.reshape(1, _DIM))

    return out.reshape(_N)
